# baseline (speedup 1.0000x reference)
import functools
import jax
import jax.numpy as jnp
from jax import lax
from jax.experimental import pallas as pl


def _walk_body(a_ref, w_ref, sel_ref):
    x = a_ref[0]
    b = x.shape[0]
    m = jnp.max(x, axis=-1, keepdims=True)
    e = jnp.exp(x - m)
    s = jnp.sum(e, axis=-1, keepdims=True)
    p = e / s
    p2 = p * p
    p4 = p2 * p2
    w = p4 * p4
    row = lax.broadcasted_iota(jnp.int32, (b, b), 0)
    col = lax.broadcasted_iota(jnp.int32, (b, b), 1)
    w = jnp.where(col > row, 0.0, w)
    w_ref[0] = w

    cur = w
    cols = []
    zero = jnp.zeros((b, 1), dtype=jnp.int32)
    cols.append(zero)
    for _ in range(15):
        mx = jnp.max(cur, axis=-1, keepdims=True)
        is_max = cur == mx
        idx = jnp.min(jnp.where(is_max, col, b), axis=-1, keepdims=True)
        cols.append(idx)
        cur = jnp.where(col == idx, -1.0, cur)
    sel_ref[0] = jnp.concatenate(cols, axis=-1)


def kernel(A_hat, layer_idx, num_query_blocks):
    del layer_idx, num_query_blocks
    batch, b, _ = A_hat.shape
    walk_state, selected = pl.pallas_call(
        _walk_body,
        grid=(batch,),
        in_specs=[pl.BlockSpec((1, b, b), lambda i: (i, 0, 0))],
        out_specs=[
            pl.BlockSpec((1, b, b), lambda i: (i, 0, 0)),
            pl.BlockSpec((1, b, 16), lambda i: (i, 0, 0)),
        ],
        out_shape=[
            jax.ShapeDtypeStruct((batch, b, b), jnp.float32),
            jax.ShapeDtypeStruct((batch, b, 16), jnp.int32),
        ],
    )(A_hat)
    return walk_state, selected


# speedup vs baseline: 5.1793x; 5.1793x over previous
import functools
import jax
import jax.numpy as jnp
from jax import lax
from jax.experimental import pallas as pl
from jax.experimental.pallas import tpu as pltpu
from jax.experimental.pallas import tpu_sc as plsc

_B = 128
_NROWS = 128 * 128
_NW = 32
_RPW = _NROWS // _NW
_R = 256


def _sc_body(a_hbm, w_hbm, sel_hbm, buf, selbuf):
    c = lax.axis_index("c")
    s = lax.axis_index("s")
    wid = s * 2 + c
    slab = wid * _RPW
    iota = lax.broadcasted_iota(jnp.int32, (16,), 0)
    fneg = -1.0 - iota.astype(jnp.float32)
    zeros16 = jnp.zeros((16,), jnp.int32)

    def merge(ak, ai, bk, bi):
        brk = lax.rev(bk, (0,))
        bri = lax.rev(bi, (0,))
        tb = (brk > ak) | ((brk == ak) & (bri < ai))
        hk = jnp.where(tb, brk, ak)
        hi = jnp.where(tb, bri, ai)
        return plsc.sort_key_val(hk, hi, descending=True)

    def chunk_body(ci, _):
        base = slab + ci * _R
        pltpu.sync_copy(a_hbm.at[pl.ds(base * _B, _R * _B)], buf)

        def row_body(r, _):
            rr = jnp.bitwise_and(base + r, _B - 1)
            off = r * _B
            v = [buf[pl.ds(off + 16 * j, 16)] for j in range(8)]
            m = jnp.maximum(jnp.maximum(jnp.maximum(v[0], v[1]),
                                        jnp.maximum(v[2], v[3])),
                            jnp.maximum(jnp.maximum(v[4], v[5]),
                                        jnp.maximum(v[6], v[7])))
            mx = jnp.max(m)
            e = [jnp.exp(vj - mx) for vj in v]
            t = (((e[0] + e[1]) + (e[2] + e[3])) +
                 ((e[4] + e[5]) + (e[6] + e[7])))
            ssum = jnp.sum(t)
            keys = []
            idxs = []
            for j in range(8):
                p = e[j] / ssum
                p2 = p * p
                p4 = p2 * p2
                p8 = p4 * p4
                gcol = iota + (16 * j)
                masked = gcol > rr
                w8 = jnp.where(masked, 0.0, p8)
                buf[pl.ds(off + 16 * j, 16)] = w8
                key = jnp.where(masked, fneg - (16.0 * j), p8)
                k_s, i_s = plsc.sort_key_val(key, gcol, descending=True)
                keys.append(k_s)
                idxs.append(i_s)
            while len(keys) > 1:
                nk, ni = [], []
                for j in range(0, len(keys), 2):
                    k2, i2 = merge(keys[j], idxs[j], keys[j + 1], idxs[j + 1])
                    nk.append(k2)
                    ni.append(i2)
                keys, idxs = nk, ni
            selbuf[pl.ds(r * 16, 16)] = zeros16
            plsc.store_scatter(selbuf, [r * 16 + 1 + iota], idxs[0],
                               mask=iota < 15)
            return 0

        lax.fori_loop(0, _R, row_body, 0)
        pltpu.sync_copy(buf, w_hbm.at[pl.ds(base * _B, _R * _B)])
        pltpu.sync_copy(selbuf, sel_hbm.at[pl.ds(base * 16, _R * 16)])
        return 0

    lax.fori_loop(0, _RPW // _R, chunk_body, 0)


_sc_kernel = pl.kernel(
    _sc_body,
    out_type=(
        jax.ShapeDtypeStruct((_NROWS * _B,), jnp.float32),
        jax.ShapeDtypeStruct((_NROWS * 16,), jnp.int32),
    ),
    mesh=plsc.VectorSubcoreMesh(core_axis_name="c", subcore_axis_name="s",
                                num_cores=2, num_subcores=16),
    scratch_types=[
        pltpu.VMEM((_R * _B,), jnp.float32),
        pltpu.VMEM((_R * 16,), jnp.int32),
    ],
    compiler_params=pltpu.CompilerParams(needs_layout_passes=False),
)


def kernel(A_hat, layer_idx, num_query_blocks):
    del layer_idx, num_query_blocks
    batch, b, _ = A_hat.shape
    w_flat, sel_flat = _sc_kernel(A_hat.reshape(-1))
    walk_state = w_flat.reshape(batch, b, b)
    selected = sel_flat.reshape(batch, b, 16)
    return walk_state, selected


# speedup vs baseline: 7.3743x; 1.4238x over previous
import functools
import jax
import jax.numpy as jnp
from jax import lax
from jax.experimental import pallas as pl
from jax.experimental.pallas import tpu as pltpu
from jax.experimental.pallas import tpu_sc as plsc

_B = 128
_NROWS = 128 * 128
_NW = 32
_RPW = _NROWS // _NW
_R = 256


def _sc_body(a_hbm, w_hbm, sel_hbm, buf, selbuf):
    c = lax.axis_index("c")
    s = lax.axis_index("s")
    wid = s * 2 + c
    slab = wid * _RPW
    iota = lax.broadcasted_iota(jnp.int32, (16,), 0)
    fneg = -1.0 - iota.astype(jnp.float32)
    zeros16 = jnp.zeros((16,), jnp.int32)

    def merge(ak, ai, bk, bi):
        brk = lax.rev(bk, (0,))
        bri = lax.rev(bi, (0,))
        tb = (brk > ak) | ((brk == ak) & (bri < ai))
        hk = jnp.where(tb, brk, ak)
        hi = jnp.where(tb, bri, ai)
        return plsc.sort_key_val(hk, hi, descending=True)

    def chunk_body(ci, _):
        base = slab + ci * _R
        pltpu.sync_copy(a_hbm.at[pl.ds(base * _B, _R * _B)], buf)

        @plsc.parallel_loop(0, _R, 1, unroll=2)
        def row_body(r):
            rr = jnp.bitwise_and(base + r, _B - 1)
            off = r * _B
            v = [buf[pl.ds(off + 16 * j, 16)] for j in range(8)]
            m = jnp.maximum(jnp.maximum(jnp.maximum(v[0], v[1]),
                                        jnp.maximum(v[2], v[3])),
                            jnp.maximum(jnp.maximum(v[4], v[5]),
                                        jnp.maximum(v[6], v[7])))
            mx = jnp.max(m)
            e = [jnp.exp(vj - mx) for vj in v]
            t = (((e[0] + e[1]) + (e[2] + e[3])) +
                 ((e[4] + e[5]) + (e[6] + e[7])))
            ssum = jnp.sum(t)
            keys = []
            idxs = []
            for j in range(8):
                p = e[j] / ssum
                p2 = p * p
                p4 = p2 * p2
                p8 = p4 * p4
                gcol = iota + (16 * j)
                masked = gcol > rr
                w8 = jnp.where(masked, 0.0, p8)
                buf[pl.ds(off + 16 * j, 16)] = w8
                key = jnp.where(masked, fneg - (16.0 * j), p8)
                k_s, i_s = plsc.sort_key_val(key, gcol, descending=True)
                keys.append(k_s)
                idxs.append(i_s)
            while len(keys) > 1:
                nk, ni = [], []
                for j in range(0, len(keys), 2):
                    k2, i2 = merge(keys[j], idxs[j], keys[j + 1], idxs[j + 1])
                    nk.append(k2)
                    ni.append(i2)
                keys, idxs = nk, ni
            selbuf[pl.ds(r * 16, 16)] = zeros16
            plsc.store_scatter(selbuf, [r * 16 + 1 + iota], idxs[0],
                               mask=iota < 15)
        pltpu.sync_copy(buf, w_hbm.at[pl.ds(base * _B, _R * _B)])
        pltpu.sync_copy(selbuf, sel_hbm.at[pl.ds(base * 16, _R * 16)])
        return 0

    lax.fori_loop(0, _RPW // _R, chunk_body, 0)


_sc_kernel = pl.kernel(
    _sc_body,
    out_type=(
        jax.ShapeDtypeStruct((_NROWS * _B,), jnp.float32),
        jax.ShapeDtypeStruct((_NROWS * 16,), jnp.int32),
    ),
    mesh=plsc.VectorSubcoreMesh(core_axis_name="c", subcore_axis_name="s",
                                num_cores=2, num_subcores=16),
    scratch_types=[
        pltpu.VMEM((_R * _B,), jnp.float32),
        pltpu.VMEM((_R * 16,), jnp.int32),
    ],
    compiler_params=pltpu.CompilerParams(needs_layout_passes=False),
)


def kernel(A_hat, layer_idx, num_query_blocks):
    del layer_idx, num_query_blocks
    batch, b, _ = A_hat.shape
    w_flat, sel_flat = _sc_kernel(A_hat.reshape(-1))
    walk_state = w_flat.reshape(batch, b, b)
    selected = sel_flat.reshape(batch, b, 16)
    return walk_state, selected


# speedup vs baseline: 7.8763x; 1.0681x over previous
import functools
import jax
import jax.numpy as jnp
from jax import lax
from jax.experimental import pallas as pl
from jax.experimental.pallas import tpu as pltpu
from jax.experimental.pallas import tpu_sc as plsc

_B = 128
_NBATCH = 128
_NW = 32
_BPW = _NBATCH // _NW
_CB = 2


def _sc_body(a_hbm, w_hbm, sel_hbm, buf, selbuf):
    c = lax.axis_index("c")
    s = lax.axis_index("s")
    wid = s * 2 + c
    batch0 = wid * _BPW
    iota = lax.broadcasted_iota(jnp.int32, (16,), 0)
    rot1 = jnp.bitwise_and(iota + 1, 15)
    negramp = [-1.0 - (iota + 16 * j).astype(jnp.float32) for j in range(8)]
    gcols = [iota + 16 * j for j in range(8)]

    def merge(ak, ai, bk, bi):
        brk = lax.rev(bk, (0,))
        bri = lax.rev(bi, (0,))
        tb = (brk > ak) | ((brk == ak) & (bri < ai))
        hk = jnp.where(tb, brk, ak)
        hi = jnp.where(tb, bri, ai)
        return plsc.sort_key_val(hk, hi, descending=True)

    def chunk_body(ci, _):
        bbase = batch0 + ci * _CB
        pltpu.sync_copy(a_hbm.at[pl.ds(bbase, _CB)], buf)

        @plsc.parallel_loop(0, _CB * _B, 1, unroll=2)
        def row_body(r):
            bi = lax.shift_right_logical(r, 7)
            ri = jnp.bitwise_and(r, _B - 1)
            v = [buf[bi, ri, pl.ds(16 * j, 16)] for j in range(8)]
            m = jnp.maximum(jnp.maximum(jnp.maximum(v[0], v[1]),
                                        jnp.maximum(v[2], v[3])),
                            jnp.maximum(jnp.maximum(v[4], v[5]),
                                        jnp.maximum(v[6], v[7])))
            mx = jnp.max(m)
            e = [jnp.exp(vj - mx) for vj in v]
            t = (((e[0] + e[1]) + (e[2] + e[3])) +
                 ((e[4] + e[5]) + (e[6] + e[7])))
            ssum = jnp.sum(t)
            keys = []
            idxs = []
            for j in range(8):
                p = e[j] / ssum
                p2 = p * p
                p4 = p2 * p2
                p8 = p4 * p4
                masked = gcols[j] > ri
                w8 = jnp.where(masked, 0.0, p8)
                buf[bi, ri, pl.ds(16 * j, 16)] = w8
                key = jnp.where(masked, negramp[j], p8)
                k_s, i_s = plsc.sort_key_val(key, gcols[j], descending=True)
                keys.append(k_s)
                idxs.append(i_s)
            while len(keys) > 1:
                nk, ni = [], []
                for j in range(0, len(keys), 2):
                    k2, i2 = merge(keys[j], idxs[j], keys[j + 1], idxs[j + 1])
                    nk.append(k2)
                    ni.append(i2)
                keys, idxs = nk, ni
            fi = jnp.where(iota == 15, 0, idxs[0])
            plsc.store_scatter(
                selbuf,
                [jnp.full((16,), bi, jnp.int32), jnp.full((16,), ri, jnp.int32),
                 rot1],
                fi)

        pltpu.sync_copy(buf, w_hbm.at[pl.ds(bbase, _CB)])
        pltpu.sync_copy(selbuf, sel_hbm.at[pl.ds(bbase, _CB)])
        return 0

    lax.fori_loop(0, _BPW // _CB, chunk_body, 0)


_sc_kernel = pl.kernel(
    _sc_body,
    out_type=(
        jax.ShapeDtypeStruct((_NBATCH, _B, _B), jnp.float32),
        jax.ShapeDtypeStruct((_NBATCH, _B, 16), jnp.int32),
    ),
    mesh=plsc.VectorSubcoreMesh(core_axis_name="c", subcore_axis_name="s",
                                num_cores=2, num_subcores=16),
    scratch_types=[
        pltpu.VMEM((_CB, _B, _B), jnp.float32),
        pltpu.VMEM((_CB, _B, 16), jnp.int32),
    ],
    compiler_params=pltpu.CompilerParams(needs_layout_passes=False),
)


def kernel(A_hat, layer_idx, num_query_blocks):
    del layer_idx, num_query_blocks
    walk_state, selected = _sc_kernel(A_hat)
    return walk_state, selected


# speedup vs baseline: 7.8893x; 1.0016x over previous
import functools
import jax
import jax.numpy as jnp
from jax import lax
from jax.experimental import pallas as pl
from jax.experimental.pallas import tpu as pltpu
from jax.experimental.pallas import tpu_sc as plsc

_B = 128
_NBATCH = 128
_NW = 32
_BPW = _NBATCH // _NW
_CB = 2


def _sc_body(a_hbm, w_hbm, sel_hbm, buf, selbuf):
    c = lax.axis_index("c")
    s = lax.axis_index("s")
    wid = s * 2 + c
    batch0 = wid * _BPW
    iota = lax.broadcasted_iota(jnp.int32, (16,), 0)
    rot1 = jnp.bitwise_and(iota + 1, 15)
    negramp = [-1.0 - (iota + 16 * j).astype(jnp.float32) for j in range(8)]
    gcols = [iota + 16 * j for j in range(8)]

    def merge(ak, ai, bk, bi):
        brk = lax.rev(bk, (0,))
        bri = lax.rev(bi, (0,))
        tb = (brk > ak) | ((brk == ak) & (bri < ai))
        hk = jnp.where(tb, brk, ak)
        hi = jnp.where(tb, bri, ai)
        return plsc.sort_key_val(hk, hi, descending=True)

    def chunk_body(ci, _):
        bbase = batch0 + ci * _CB
        pltpu.sync_copy(a_hbm.at[pl.ds(bbase, _CB)], buf)

        @plsc.parallel_loop(0, _CB * _B, 1, unroll=2)
        def row_body(r):
            bi = lax.shift_right_logical(r, 7)
            ri = jnp.bitwise_and(r, _B - 1)
            v = [buf[bi, ri, pl.ds(16 * j, 16)] for j in range(8)]
            m = jnp.maximum(jnp.maximum(jnp.maximum(v[0], v[1]),
                                        jnp.maximum(v[2], v[3])),
                            jnp.maximum(jnp.maximum(v[4], v[5]),
                                        jnp.maximum(v[6], v[7])))
            mx = jnp.max(m)
            e = [jnp.exp(vj - mx) for vj in v]
            t = (((e[0] + e[1]) + (e[2] + e[3])) +
                 ((e[4] + e[5]) + (e[6] + e[7])))
            ssum = jnp.sum(t)
            keys = []
            idxs = []
            for j in range(8):
                p = e[j] / ssum
                p2 = p * p
                p4 = p2 * p2
                p8 = p4 * p4
                masked = gcols[j] > ri
                w8 = jnp.where(masked, 0.0, p8)
                buf[bi, ri, pl.ds(16 * j, 16)] = w8
                key = jnp.where(masked, negramp[j], p8)
                k_s, i_s = plsc.sort_key_val(key, gcols[j], descending=True)
                keys.append(k_s)
                idxs.append(i_s)
            while len(keys) > 1:
                nk, ni = [], []
                for j in range(0, len(keys), 2):
                    k2, i2 = merge(keys[j], idxs[j], keys[j + 1], idxs[j + 1])
                    nk.append(k2)
                    ni.append(i2)
                keys, idxs = nk, ni
            fi = jnp.where(iota == 15, 0, idxs[0])
            plsc.store_scatter(
                selbuf,
                [jnp.full((16,), bi, jnp.int32), jnp.full((16,), ri, jnp.int32),
                 rot1],
                fi)

        pltpu.sync_copy(buf, w_hbm.at[pl.ds(bbase, _CB)])
        pltpu.sync_copy(selbuf, sel_hbm.at[pl.ds(bbase, _CB)])
        return 0

    lax.fori_loop(0, _BPW // _CB, chunk_body, 0)


_sc_kernel = pl.kernel(
    _sc_body,
    out_type=(
        jax.ShapeDtypeStruct((_NBATCH, _B, _B), jnp.float32),
        jax.ShapeDtypeStruct((_NBATCH, _B, 16), jnp.int32),
    ),
    mesh=plsc.VectorSubcoreMesh(core_axis_name="c", subcore_axis_name="s",
                                num_cores=2, num_subcores=16),
    scratch_types=[
        pltpu.VMEM((_CB, _B, _B), jnp.float32),
        pltpu.VMEM((_CB, _B, 16), jnp.int32),
    ],
    compiler_params=pltpu.CompilerParams(needs_layout_passes=False,
                                         use_tc_tiling_on_sc=True),
)


def kernel(A_hat, layer_idx, num_query_blocks):
    del layer_idx, num_query_blocks
    walk_state, selected = _sc_kernel(A_hat)
    return walk_state, selected
